# baseline (speedup 1.0000x reference)
import functools

import jax
import jax.numpy as jnp
from jax import lax
from jax.experimental import pallas as pl
from jax.experimental.pallas import tpu as pltpu
from jax.experimental.pallas import tpu_sc as plsc

_NC = 2
_NS = 16
_NW = _NC * _NS


def _sc_gather_pairs(pair_table, half_idx):
    n_idx = half_idx.shape[0]
    d = pair_table.shape[1]
    b_per_w = n_idx // _NW
    n_chunks = 4
    chunk = b_per_w // n_chunks
    mesh = plsc.VectorSubcoreMesh(core_axis_name="c", subcore_axis_name="s")

    @functools.partial(
        pl.kernel,
        mesh=mesh,
        out_type=jax.ShapeDtypeStruct((n_idx, d), pair_table.dtype),
        scratch_types=[
            pltpu.VMEM((b_per_w,), jnp.int32),
            pltpu.VMEM((chunk, d), jnp.float32),
            pltpu.VMEM((chunk, d), jnp.float32),
            pltpu.SemaphoreType.DMA,
            pltpu.SemaphoreType.DMA,
        ],
    )
    def gather_kernel(tab_hbm, idx_hbm, out_hbm, idx_v, rows_a, rows_b, sem_a,
                      sem_b):
        wid = lax.axis_index("s") * _NC + lax.axis_index("c")
        base = wid * b_per_w
        pltpu.sync_copy(idx_hbm.at[pl.ds(base, b_per_w)], idx_v)
        bufs = (rows_a, rows_b)
        sems = (sem_a, sem_b)
        cps = {}
        for i in range(2):
            cps[i] = pltpu.async_copy(
                tab_hbm.at[idx_v.at[pl.ds(i * chunk, chunk)]], bufs[i],
                sems[i])
        for i in range(n_chunks):
            cps[i].wait()
            pltpu.sync_copy(bufs[i % 2],
                            out_hbm.at[pl.ds(base + i * chunk, chunk)])
            if i + 2 < n_chunks:
                cps[i + 2] = pltpu.async_copy(
                    tab_hbm.at[idx_v.at[pl.ds((i + 2) * chunk, chunk)]],
                    bufs[i % 2], sems[i % 2])

    return gather_kernel(pair_table, half_idx)


def _mlp_body(x_ref, p_ref, w1_ref, b1_ref, w2_ref, b2_ref, w3_ref, b3_ref,
              w4_ref, b4_ref, o_ref):
    x = x_ref[...]
    d = x.shape[1] // 4
    pu = p_ref[:, 0:1] & 1
    pv = p_ref[:, 1:2] & 1
    u = jnp.where(pu == 0, x[:, 0:d], x[:, d:2 * d])
    v = jnp.where(pv == 0, x[:, 2 * d:3 * d], x[:, 3 * d:4 * d])
    mf = u * v
    mlp = jnp.concatenate([u, v], axis=1)
    h = jnp.maximum(
        jnp.dot(mlp, w1_ref[...], preferred_element_type=jnp.float32)
        + b1_ref[...], 0.0)
    h = jnp.maximum(
        jnp.dot(h, w2_ref[...], preferred_element_type=jnp.float32)
        + b2_ref[...], 0.0)
    h = jnp.maximum(
        jnp.dot(h, w3_ref[...], preferred_element_type=jnp.float32)
        + b3_ref[...], 0.0)
    nh = h.shape[1]
    out = (jnp.dot(h, w4_ref[:nh, :], preferred_element_type=jnp.float32)
           + jnp.dot(mf, w4_ref[nh:, :], preferred_element_type=jnp.float32)
           + b4_ref[...])
    o_ref[...] = out


def kernel(interaction_pairs, table, W1, b1, W2, b2, W3, b3, W4, b4):
    batch = interaction_pairs.shape[0]
    d = table.shape[1]
    pair_table = table.reshape(-1, 2 * d)
    half_idx = interaction_pairs.reshape(-1) >> 1

    gathered = _sc_gather_pairs(pair_table, half_idx)
    x = gathered.reshape(batch, 4 * d)

    blk = 2048
    grid = (batch // blk,)
    full = lambda shape: pl.BlockSpec(shape, lambda i: (0, 0))
    out = pl.pallas_call(
        _mlp_body,
        grid=grid,
        in_specs=[
            pl.BlockSpec((blk, 4 * d), lambda i: (i, 0)),
            pl.BlockSpec((blk, 2), lambda i: (i, 0)),
            full(W1.shape),
            full((1, W1.shape[1])),
            full(W2.shape),
            full((1, W2.shape[1])),
            full(W3.shape),
            full((1, W3.shape[1])),
            full(W4.shape),
            full((1, 1)),
        ],
        out_specs=pl.BlockSpec((blk, 1), lambda i: (i, 0)),
        out_shape=jax.ShapeDtypeStruct((batch, 1), jnp.float32),
    )(x, interaction_pairs, W1, b1.reshape(1, -1), W2, b2.reshape(1, -1),
      W3, b3.reshape(1, -1), W4, b4.reshape(1, 1))
    return out[:, 0]


# speedup vs baseline: 1.0097x; 1.0097x over previous
import functools

import jax
import jax.numpy as jnp
from jax import lax
from jax.experimental import pallas as pl
from jax.experimental.pallas import tpu as pltpu
from jax.experimental.pallas import tpu_sc as plsc

_NC = 2
_NS = 16
_NW = _NC * _NS


def _pack_body(lo_ref, hi_ref, o_ref):
    o_ref[...] = jnp.concatenate([lo_ref[...], hi_ref[...]], axis=1)


def _pack_pairs(table):
    v, d = table.shape
    blk = 4000
    n_blocks = (v // 2) // blk
    return pl.pallas_call(
        _pack_body,
        grid=(n_blocks,),
        in_specs=[
            pl.BlockSpec((blk, d), lambda i: (i, 0)),
            pl.BlockSpec((blk, d), lambda i, n=n_blocks: (n + i, 0)),
        ],
        out_specs=pl.BlockSpec((blk, 2 * d), lambda i: (i, 0)),
        out_shape=jax.ShapeDtypeStruct((v // 2, 2 * d), table.dtype),
    )(table, table)


def _sc_gather_pairs(pair_table, half_idx):
    n_idx = half_idx.shape[0]
    d = pair_table.shape[1]
    b_per_w = n_idx // _NW
    n_chunks = 4
    chunk = b_per_w // n_chunks
    mesh = plsc.VectorSubcoreMesh(core_axis_name="c", subcore_axis_name="s")

    @functools.partial(
        pl.kernel,
        mesh=mesh,
        out_type=jax.ShapeDtypeStruct((n_idx, d), pair_table.dtype),
        scratch_types=[
            pltpu.VMEM((b_per_w,), jnp.int32),
            pltpu.VMEM((chunk, d), jnp.float32),
            pltpu.VMEM((chunk, d), jnp.float32),
            pltpu.SemaphoreType.DMA,
            pltpu.SemaphoreType.DMA,
        ],
    )
    def gather_kernel(tab_hbm, idx_hbm, out_hbm, idx_v, rows_a, rows_b,
                      sem_a, sem_b):
        wid = lax.axis_index("s") * _NC + lax.axis_index("c")
        base = wid * b_per_w
        pltpu.sync_copy(idx_hbm.at[pl.ds(base, b_per_w)], idx_v)
        bufs = (rows_a, rows_b)
        sems = (sem_a, sem_b)
        cps = {}
        for i in range(2):
            cps[i] = pltpu.async_copy(
                tab_hbm.at[idx_v.at[pl.ds(i * chunk, chunk)]], bufs[i],
                sems[i])
        for i in range(n_chunks):
            cps[i].wait()
            pltpu.sync_copy(bufs[i % 2],
                            out_hbm.at[pl.ds(base + i * chunk, chunk)])
            if i + 2 < n_chunks:
                cps[i + 2] = pltpu.async_copy(
                    tab_hbm.at[idx_v.at[pl.ds((i + 2) * chunk, chunk)]],
                    bufs[i % 2], sems[i % 2])

    return gather_kernel(pair_table, half_idx)


def _mlp_body(half_v, x_ref, p_ref, w1_ref, b1_ref, w2_ref, b2_ref, w3_ref,
              b3_ref, w4_ref, b4_ref, o_ref):
    x = x_ref[...]
    d = x.shape[1] // 4
    pu = p_ref[:, 0:1]
    pv = p_ref[:, 1:2]
    u = jnp.where(pu < half_v, x[:, 0:d], x[:, d:2 * d])
    v = jnp.where(pv < half_v, x[:, 2 * d:3 * d], x[:, 3 * d:4 * d])
    mf = u * v
    mlp = jnp.concatenate([u, v], axis=1)
    h = jnp.maximum(
        jnp.dot(mlp, w1_ref[...], preferred_element_type=jnp.float32)
        + b1_ref[...], 0.0)
    h = jnp.maximum(
        jnp.dot(h, w2_ref[...], preferred_element_type=jnp.float32)
        + b2_ref[...], 0.0)
    h = jnp.maximum(
        jnp.dot(h, w3_ref[...], preferred_element_type=jnp.float32)
        + b3_ref[...], 0.0)
    nh = h.shape[1]
    out = (jnp.dot(h, w4_ref[:nh, :], preferred_element_type=jnp.float32)
           + jnp.dot(mf, w4_ref[nh:, :], preferred_element_type=jnp.float32)
           + b4_ref[...])
    o_ref[...] = out


def kernel(interaction_pairs, table, W1, b1, W2, b2, W3, b3, W4, b4):
    batch = interaction_pairs.shape[0]
    d = table.shape[1]
    half_v = table.shape[0] // 2
    pair_table = _pack_pairs(table)
    flat_idx = interaction_pairs.reshape(-1)
    half_idx = jnp.where(flat_idx < half_v, flat_idx, flat_idx - half_v)

    gathered = _sc_gather_pairs(pair_table, half_idx)
    x = gathered.reshape(batch, 4 * d)

    blk = 2048
    grid = (batch // blk,)
    full = lambda shape: pl.BlockSpec(shape, lambda i: (0, 0))
    out = pl.pallas_call(
        functools.partial(_mlp_body, half_v),
        grid=grid,
        in_specs=[
            pl.BlockSpec((blk, 4 * d), lambda i: (i, 0)),
            pl.BlockSpec((blk, 2), lambda i: (i, 0)),
            full(W1.shape),
            full((1, W1.shape[1])),
            full(W2.shape),
            full((1, W2.shape[1])),
            full(W3.shape),
            full((1, W3.shape[1])),
            full(W4.shape),
            full((1, 1)),
        ],
        out_specs=pl.BlockSpec((blk, 1), lambda i: (i, 0)),
        out_shape=jax.ShapeDtypeStruct((batch, 1), jnp.float32),
    )(x, interaction_pairs, W1, b1.reshape(1, -1), W2, b2.reshape(1, -1),
      W3, b3.reshape(1, -1), W4, b4.reshape(1, 1))
    return out[:, 0]


# speedup vs baseline: 1.0104x; 1.0007x over previous
import functools

import jax
import jax.numpy as jnp
from jax import lax
from jax.experimental import pallas as pl
from jax.experimental.pallas import tpu as pltpu
from jax.experimental.pallas import tpu_sc as plsc

_NC = 2
_NS = 16
_NW = _NC * _NS


def _pack_body(half_v, tab_ref, o_ref, lo0, lo1, hi0, hi1, sl0, sl1, sh0,
               sh1):
    i = pl.program_id(0)
    n = pl.num_programs(0)
    blk = o_ref.shape[0]

    def issue(j, lo, hi, sl, sh):
        pltpu.make_async_copy(tab_ref.at[pl.ds(j * blk, blk), :], lo,
                              sl).start()
        pltpu.make_async_copy(
            tab_ref.at[pl.ds(half_v + j * blk, blk), :], hi, sh).start()

    def finish(lo, hi, sl, sh):
        pltpu.make_async_copy(tab_ref.at[pl.ds(0, blk), :], lo, sl).wait()
        pltpu.make_async_copy(tab_ref.at[pl.ds(0, blk), :], hi, sh).wait()
        o_ref[...] = jnp.concatenate([lo[...], hi[...]], axis=1)

    even = (i % 2) == 0

    @pl.when(i == 0)
    def _():
        issue(0, lo0, hi0, sl0, sh0)

    @pl.when(jnp.logical_and(even, i + 1 < n))
    def _():
        issue(i + 1, lo1, hi1, sl1, sh1)

    @pl.when(jnp.logical_and(jnp.logical_not(even), i + 1 < n))
    def _():
        issue(i + 1, lo0, hi0, sl0, sh0)

    @pl.when(even)
    def _():
        finish(lo0, hi0, sl0, sh0)

    @pl.when(jnp.logical_not(even))
    def _():
        finish(lo1, hi1, sl1, sh1)


def _pack_pairs(table):
    v, d = table.shape
    half_v = v // 2
    blk = 4000
    n_blocks = half_v // blk
    buf = pltpu.VMEM((blk, d), jnp.float32)
    return pl.pallas_call(
        functools.partial(_pack_body, half_v),
        grid=(n_blocks,),
        in_specs=[pl.BlockSpec(memory_space=pl.ANY)],
        out_specs=pl.BlockSpec((blk, 2 * d), lambda i: (i, 0)),
        out_shape=jax.ShapeDtypeStruct((half_v, 2 * d), table.dtype),
        scratch_shapes=[buf, buf, buf, buf,
                        pltpu.SemaphoreType.DMA, pltpu.SemaphoreType.DMA,
                        pltpu.SemaphoreType.DMA, pltpu.SemaphoreType.DMA],
    )(table)


def _sc_gather_pairs(pair_table, half_idx):
    n_idx = half_idx.shape[0]
    d = pair_table.shape[1]
    b_per_w = n_idx // _NW
    n_chunks = 4
    chunk = b_per_w // n_chunks
    mesh = plsc.VectorSubcoreMesh(core_axis_name="c", subcore_axis_name="s")

    @functools.partial(
        pl.kernel,
        mesh=mesh,
        out_type=jax.ShapeDtypeStruct((n_idx, d), pair_table.dtype),
        scratch_types=[
            pltpu.VMEM((b_per_w,), jnp.int32),
            pltpu.VMEM((chunk, d), jnp.float32),
            pltpu.VMEM((chunk, d), jnp.float32),
            pltpu.SemaphoreType.DMA,
            pltpu.SemaphoreType.DMA,
        ],
    )
    def gather_kernel(tab_hbm, idx_hbm, out_hbm, idx_v, rows_a, rows_b,
                      sem_a, sem_b):
        wid = lax.axis_index("s") * _NC + lax.axis_index("c")
        base = wid * b_per_w
        pltpu.sync_copy(idx_hbm.at[pl.ds(base, b_per_w)], idx_v)
        bufs = (rows_a, rows_b)
        sems = (sem_a, sem_b)
        cps = {}
        for i in range(2):
            cps[i] = pltpu.async_copy(
                tab_hbm.at[idx_v.at[pl.ds(i * chunk, chunk)]], bufs[i],
                sems[i])
        for i in range(n_chunks):
            cps[i].wait()
            pltpu.sync_copy(bufs[i % 2],
                            out_hbm.at[pl.ds(base + i * chunk, chunk)])
            if i + 2 < n_chunks:
                cps[i + 2] = pltpu.async_copy(
                    tab_hbm.at[idx_v.at[pl.ds((i + 2) * chunk, chunk)]],
                    bufs[i % 2], sems[i % 2])

    return gather_kernel(pair_table, half_idx)


def _mlp_body(half_v, x_ref, p_ref, w1_ref, b1_ref, w2_ref, b2_ref, w3_ref,
              b3_ref, w4_ref, b4_ref, o_ref):
    x = x_ref[...]
    d = x.shape[1] // 4
    pu = p_ref[:, 0:1]
    pv = p_ref[:, 1:2]
    u = jnp.where(pu < half_v, x[:, 0:d], x[:, d:2 * d])
    v = jnp.where(pv < half_v, x[:, 2 * d:3 * d], x[:, 3 * d:4 * d])
    mf = u * v
    mlp = jnp.concatenate([u, v], axis=1)
    h = jnp.maximum(
        jnp.dot(mlp, w1_ref[...], preferred_element_type=jnp.float32)
        + b1_ref[...], 0.0)
    h = jnp.maximum(
        jnp.dot(h, w2_ref[...], preferred_element_type=jnp.float32)
        + b2_ref[...], 0.0)
    h = jnp.maximum(
        jnp.dot(h, w3_ref[...], preferred_element_type=jnp.float32)
        + b3_ref[...], 0.0)
    nh = h.shape[1]
    out = (jnp.dot(h, w4_ref[:nh, :], preferred_element_type=jnp.float32)
           + jnp.dot(mf, w4_ref[nh:, :], preferred_element_type=jnp.float32)
           + b4_ref[...])
    o_ref[...] = out


def kernel(interaction_pairs, table, W1, b1, W2, b2, W3, b3, W4, b4):
    batch = interaction_pairs.shape[0]
    d = table.shape[1]
    half_v = table.shape[0] // 2
    pair_table = _pack_pairs(table)
    flat_idx = interaction_pairs.reshape(-1)
    half_idx = jnp.where(flat_idx < half_v, flat_idx, flat_idx - half_v)

    gathered = _sc_gather_pairs(pair_table, half_idx)
    x = gathered.reshape(batch, 4 * d)

    blk = 2048
    grid = (batch // blk,)
    full = lambda shape: pl.BlockSpec(shape, lambda i: (0, 0))
    out = pl.pallas_call(
        functools.partial(_mlp_body, half_v),
        grid=grid,
        in_specs=[
            pl.BlockSpec((blk, 4 * d), lambda i: (i, 0)),
            pl.BlockSpec((blk, 2), lambda i: (i, 0)),
            full(W1.shape),
            full((1, W1.shape[1])),
            full(W2.shape),
            full((1, W2.shape[1])),
            full(W3.shape),
            full((1, W3.shape[1])),
            full(W4.shape),
            full((1, 1)),
        ],
        out_specs=pl.BlockSpec((blk, 1), lambda i: (i, 0)),
        out_shape=jax.ShapeDtypeStruct((batch, 1), jnp.float32),
    )(x, interaction_pairs, W1, b1.reshape(1, -1), W2, b2.reshape(1, -1),
      W3, b3.reshape(1, -1), W4, b4.reshape(1, 1))
    return out[:, 0]


# speedup vs baseline: 1.8695x; 1.8503x over previous
import functools

import jax
import jax.numpy as jnp
from jax import lax
from jax.experimental import pallas as pl
from jax.experimental.pallas import tpu as pltpu
from jax.experimental.pallas import tpu_sc as plsc

_NC = 2
_NS = 16
_NW = _NC * _NS
_B = 4096


def _pack_body(t_ref, o_ref):
    x = t_ref[...]
    b = x.shape[1] // 2
    lo = jnp.transpose(x[:, :b])
    hi = jnp.transpose(x[:, b:])
    o_ref[...] = jnp.concatenate([lo, hi], axis=1)


def _pack_pairs(table_t):
    d, v = table_t.shape
    n_blocks = (v + 2 * _B - 1) // (2 * _B)
    n_rows = n_blocks * _B
    return pl.pallas_call(
        _pack_body,
        grid=(n_blocks,),
        in_specs=[pl.BlockSpec((d, 2 * _B), lambda k: (0, k))],
        out_specs=pl.BlockSpec((_B, 2 * d), lambda k: (k, 0)),
        out_shape=jax.ShapeDtypeStruct((n_rows, 2 * d), table_t.dtype),
    )(table_t)


def _sc_gather_pairs(pair_table, pair_idx):
    n_idx = pair_idx.shape[0]
    d = pair_table.shape[1]
    b_per_w = n_idx // _NW
    n_chunks = 4
    chunk = b_per_w // n_chunks
    mesh = plsc.VectorSubcoreMesh(core_axis_name="c", subcore_axis_name="s")

    @functools.partial(
        pl.kernel,
        mesh=mesh,
        out_type=jax.ShapeDtypeStruct((n_idx, d), pair_table.dtype),
        scratch_types=[
            pltpu.VMEM((b_per_w,), jnp.int32),
            pltpu.VMEM((chunk, d), jnp.float32),
            pltpu.VMEM((chunk, d), jnp.float32),
            pltpu.SemaphoreType.DMA,
            pltpu.SemaphoreType.DMA,
        ],
    )
    def gather_kernel(tab_hbm, idx_hbm, out_hbm, idx_v, rows_a, rows_b,
                      sem_a, sem_b):
        wid = lax.axis_index("s") * _NC + lax.axis_index("c")
        base = wid * b_per_w
        pltpu.sync_copy(idx_hbm.at[pl.ds(base, b_per_w)], idx_v)
        bufs = (rows_a, rows_b)
        sems = (sem_a, sem_b)
        cps = {}
        for i in range(2):
            cps[i] = pltpu.async_copy(
                tab_hbm.at[idx_v.at[pl.ds(i * chunk, chunk)]], bufs[i],
                sems[i])
        for i in range(n_chunks):
            cps[i].wait()
            pltpu.sync_copy(bufs[i % 2],
                            out_hbm.at[pl.ds(base + i * chunk, chunk)])
            if i + 2 < n_chunks:
                cps[i + 2] = pltpu.async_copy(
                    tab_hbm.at[idx_v.at[pl.ds((i + 2) * chunk, chunk)]],
                    bufs[i % 2], sems[i % 2])

    return gather_kernel(pair_table, pair_idx)


def _mlp_body(x_ref, h_ref, w1_ref, b1_ref, w2_ref, b2_ref, w3_ref, b3_ref,
              w4_ref, b4_ref, o_ref):
    x = x_ref[...]
    d = x.shape[1] // 4
    hu = h_ref[:, 0:1]
    hv = h_ref[:, 1:2]
    u = jnp.where(hu == 0, x[:, 0:d], x[:, d:2 * d])
    v = jnp.where(hv == 0, x[:, 2 * d:3 * d], x[:, 3 * d:4 * d])
    mf = u * v
    mlp = jnp.concatenate([u, v], axis=1)
    h = jnp.maximum(
        jnp.dot(mlp, w1_ref[...], preferred_element_type=jnp.float32)
        + b1_ref[...], 0.0)
    h = jnp.maximum(
        jnp.dot(h, w2_ref[...], preferred_element_type=jnp.float32)
        + b2_ref[...], 0.0)
    h = jnp.maximum(
        jnp.dot(h, w3_ref[...], preferred_element_type=jnp.float32)
        + b3_ref[...], 0.0)
    nh = h.shape[1]
    out = (jnp.dot(h, w4_ref[:nh, :], preferred_element_type=jnp.float32)
           + jnp.dot(mf, w4_ref[nh:, :], preferred_element_type=jnp.float32)
           + b4_ref[...])
    o_ref[...] = out


def kernel(interaction_pairs, table, W1, b1, W2, b2, W3, b3, W4, b4):
    batch = interaction_pairs.shape[0]
    d = table.shape[1]

    pair_table = _pack_pairs(table.T)

    pair_idx = ((interaction_pairs // (2 * _B)) * _B
                + (interaction_pairs % _B)).reshape(-1)
    halves = (interaction_pairs // _B) & 1

    gathered = _sc_gather_pairs(pair_table, pair_idx)
    x = gathered.reshape(batch, 4 * d)

    blk = 2048
    grid = (batch // blk,)
    full = lambda shape: pl.BlockSpec(shape, lambda i: (0, 0))
    out = pl.pallas_call(
        _mlp_body,
        grid=grid,
        in_specs=[
            pl.BlockSpec((blk, 4 * d), lambda i: (i, 0)),
            pl.BlockSpec((blk, 2), lambda i: (i, 0)),
            full(W1.shape),
            full((1, W1.shape[1])),
            full(W2.shape),
            full((1, W2.shape[1])),
            full(W3.shape),
            full((1, W3.shape[1])),
            full(W4.shape),
            full((1, 1)),
        ],
        out_specs=pl.BlockSpec((blk, 1), lambda i: (i, 0)),
        out_shape=jax.ShapeDtypeStruct((batch, 1), jnp.float32),
    )(x, halves, W1, b1.reshape(1, -1), W2, b2.reshape(1, -1),
      W3, b3.reshape(1, -1), W4, b4.reshape(1, 1))
    return out[:, 0]


# speedup vs baseline: 2.2618x; 1.2098x over previous
import functools

import jax
import jax.numpy as jnp
from jax import lax
from jax.experimental import pallas as pl
from jax.experimental.pallas import tpu as pltpu
from jax.experimental.pallas import tpu_sc as plsc

_NC = 2
_NS = 16
_NW = _NC * _NS
_B = 4096


def _pack_body(t_ref, o_ref):
    x = t_ref[...]
    b = x.shape[1] // 2
    y = jnp.concatenate([x[:, :b], x[:, b:]], axis=0)
    o_ref[...] = jnp.transpose(y)


def _pack_pairs(table_t):
    d, v = table_t.shape
    n_blocks = (v + 2 * _B - 1) // (2 * _B)
    n_rows = n_blocks * _B
    return pl.pallas_call(
        _pack_body,
        grid=(n_blocks,),
        in_specs=[pl.BlockSpec((d, 2 * _B), lambda k: (0, k))],
        out_specs=pl.BlockSpec((_B, 2 * d), lambda k: (k, 0)),
        out_shape=jax.ShapeDtypeStruct((n_rows, 2 * d), table_t.dtype),
    )(table_t)


def _sc_gather_pairs(pair_table, pair_idx):
    n_idx = pair_idx.shape[0]
    d = pair_table.shape[1]
    b_per_w = n_idx // _NW
    n_chunks = 4
    chunk = b_per_w // n_chunks
    mesh = plsc.VectorSubcoreMesh(core_axis_name="c", subcore_axis_name="s")

    @functools.partial(
        pl.kernel,
        mesh=mesh,
        out_type=jax.ShapeDtypeStruct((n_idx, d), pair_table.dtype),
        scratch_types=[
            pltpu.VMEM((b_per_w,), jnp.int32),
            pltpu.VMEM((chunk, d), jnp.float32),
            pltpu.VMEM((chunk, d), jnp.float32),
            pltpu.SemaphoreType.DMA,
            pltpu.SemaphoreType.DMA,
        ],
    )
    def gather_kernel(tab_hbm, idx_hbm, out_hbm, idx_v, rows_a, rows_b,
                      sem_a, sem_b):
        wid = lax.axis_index("s") * _NC + lax.axis_index("c")
        base = wid * b_per_w
        pltpu.sync_copy(idx_hbm.at[pl.ds(base, b_per_w)], idx_v)
        bufs = (rows_a, rows_b)
        sems = (sem_a, sem_b)
        cps = {}
        for i in range(2):
            cps[i] = pltpu.async_copy(
                tab_hbm.at[idx_v.at[pl.ds(i * chunk, chunk)]], bufs[i],
                sems[i])
        for i in range(n_chunks):
            cps[i].wait()
            pltpu.sync_copy(bufs[i % 2],
                            out_hbm.at[pl.ds(base + i * chunk, chunk)])
            if i + 2 < n_chunks:
                cps[i + 2] = pltpu.async_copy(
                    tab_hbm.at[idx_v.at[pl.ds((i + 2) * chunk, chunk)]],
                    bufs[i % 2], sems[i % 2])

    return gather_kernel(pair_table, pair_idx)


def _mlp_body(x_ref, h_ref, w1_ref, b1_ref, w2_ref, b2_ref, w3_ref, b3_ref,
              w4_ref, b4_ref, o_ref):
    x = x_ref[...]
    d = x.shape[1] // 4
    hu = h_ref[:, 0:1]
    hv = h_ref[:, 1:2]
    u = jnp.where(hu == 0, x[:, 0:d], x[:, d:2 * d])
    v = jnp.where(hv == 0, x[:, 2 * d:3 * d], x[:, 3 * d:4 * d])
    mf = u * v
    mlp = jnp.concatenate([u, v], axis=1)
    h = jnp.maximum(
        jnp.dot(mlp, w1_ref[...], preferred_element_type=jnp.float32)
        + b1_ref[...], 0.0)
    h = jnp.maximum(
        jnp.dot(h, w2_ref[...], preferred_element_type=jnp.float32)
        + b2_ref[...], 0.0)
    h = jnp.maximum(
        jnp.dot(h, w3_ref[...], preferred_element_type=jnp.float32)
        + b3_ref[...], 0.0)
    nh = h.shape[1]
    out = (jnp.dot(h, w4_ref[:nh, :], preferred_element_type=jnp.float32)
           + jnp.dot(mf, w4_ref[nh:, :], preferred_element_type=jnp.float32)
           + b4_ref[...])
    o_ref[...] = out


def kernel(interaction_pairs, table, W1, b1, W2, b2, W3, b3, W4, b4):
    batch = interaction_pairs.shape[0]
    d = table.shape[1]

    pair_table = _pack_pairs(table.T)

    pair_idx = ((interaction_pairs // (2 * _B)) * _B
                + (interaction_pairs % _B)).reshape(-1)
    halves = (interaction_pairs // _B) & 1

    gathered = _sc_gather_pairs(pair_table, pair_idx)
    x = gathered.reshape(batch, 4 * d)

    blk = 2048
    grid = (batch // blk,)
    full = lambda shape: pl.BlockSpec(shape, lambda i: (0, 0))
    out = pl.pallas_call(
        _mlp_body,
        grid=grid,
        in_specs=[
            pl.BlockSpec((blk, 4 * d), lambda i: (i, 0)),
            pl.BlockSpec((blk, 2), lambda i: (i, 0)),
            full(W1.shape),
            full((1, W1.shape[1])),
            full(W2.shape),
            full((1, W2.shape[1])),
            full(W3.shape),
            full((1, W3.shape[1])),
            full(W4.shape),
            full((1, 1)),
        ],
        out_specs=pl.BlockSpec((blk, 1), lambda i: (i, 0)),
        out_shape=jax.ShapeDtypeStruct((batch, 1), jnp.float32),
    )(x, halves, W1, b1.reshape(1, -1), W2, b2.reshape(1, -1),
      W3, b3.reshape(1, -1), W4, b4.reshape(1, 1))
    return out[:, 0]


# speedup vs baseline: 2.9195x; 1.2908x over previous
import functools

import jax
import jax.numpy as jnp
from jax import lax
from jax.experimental import pallas as pl
from jax.experimental.pallas import tpu as pltpu
from jax.experimental.pallas import tpu_sc as plsc

_NC = 2
_NS = 16
_NW = _NC * _NS
_B = 8192


def _pack_body(t_ref, o_ref):
    x = t_ref[...]
    b = x.shape[1] // 2
    y = jnp.concatenate([x[:, :b], x[:, b:]], axis=0)
    o_ref[...] = jnp.transpose(y)


def _pack_pairs(table_t):
    d, v = table_t.shape
    n_blocks = (v + 2 * _B - 1) // (2 * _B)
    n_rows = n_blocks * _B
    return pl.pallas_call(
        _pack_body,
        grid=(n_blocks,),
        in_specs=[pl.BlockSpec((d, 2 * _B), lambda k: (0, k))],
        out_specs=pl.BlockSpec((_B, 2 * d), lambda k: (k, 0)),
        out_shape=jax.ShapeDtypeStruct((n_rows, 2 * d), table_t.dtype),
    )(table_t)


def _sc_gather_pairs(pair_table, pair_idx):
    n_idx = pair_idx.shape[0]
    d = pair_table.shape[1]
    b_per_w = n_idx // _NW
    n_chunks = 4
    chunk = b_per_w // n_chunks
    mesh = plsc.VectorSubcoreMesh(core_axis_name="c", subcore_axis_name="s")

    @functools.partial(
        pl.kernel,
        mesh=mesh,
        out_type=jax.ShapeDtypeStruct((n_idx, d), pair_table.dtype),
        scratch_types=[
            pltpu.VMEM((b_per_w,), jnp.int32),
            pltpu.VMEM((chunk, d), jnp.float32),
            pltpu.VMEM((chunk, d), jnp.float32),
            pltpu.SemaphoreType.DMA,
            pltpu.SemaphoreType.DMA,
        ],
    )
    def gather_kernel(tab_hbm, idx_hbm, out_hbm, idx_v, rows_a, rows_b,
                      sem_a, sem_b):
        wid = lax.axis_index("s") * _NC + lax.axis_index("c")
        base = wid * b_per_w
        pltpu.sync_copy(idx_hbm.at[pl.ds(base, b_per_w)], idx_v)
        bufs = (rows_a, rows_b)
        sems = (sem_a, sem_b)
        cps = {}
        for i in range(2):
            cps[i] = pltpu.async_copy(
                tab_hbm.at[idx_v.at[pl.ds(i * chunk, chunk)]], bufs[i],
                sems[i])
        for i in range(n_chunks):
            cps[i].wait()
            pltpu.sync_copy(bufs[i % 2],
                            out_hbm.at[pl.ds(base + i * chunk, chunk)])
            if i + 2 < n_chunks:
                cps[i + 2] = pltpu.async_copy(
                    tab_hbm.at[idx_v.at[pl.ds((i + 2) * chunk, chunk)]],
                    bufs[i % 2], sems[i % 2])

    return gather_kernel(pair_table, pair_idx)


def _mlp_body(xu_ref, xv_ref, h_ref, w1_ref, b1_ref, w2_ref, b2_ref, w3_ref,
              b3_ref, w4_ref, b4_ref, o_ref):
    xu = xu_ref[...]
    xv = xv_ref[...]
    d = xu.shape[1] // 2
    hu = h_ref[:, 0:1]
    hv = h_ref[:, 1:2]
    u = jnp.where(hu == 0, xu[:, 0:d], xu[:, d:2 * d])
    v = jnp.where(hv == 0, xv[:, 0:d], xv[:, d:2 * d])
    mf = u * v
    mlp = jnp.concatenate([u, v], axis=1)
    h = jnp.maximum(
        jnp.dot(mlp, w1_ref[...], preferred_element_type=jnp.float32)
        + b1_ref[...], 0.0)
    h = jnp.maximum(
        jnp.dot(h, w2_ref[...], preferred_element_type=jnp.float32)
        + b2_ref[...], 0.0)
    h = jnp.maximum(
        jnp.dot(h, w3_ref[...], preferred_element_type=jnp.float32)
        + b3_ref[...], 0.0)
    nh = h.shape[1]
    out = (jnp.dot(h, w4_ref[:nh, :], preferred_element_type=jnp.float32)
           + jnp.dot(mf, w4_ref[nh:, :], preferred_element_type=jnp.float32)
           + b4_ref[...])
    o_ref[...] = out


def kernel(interaction_pairs, table, W1, b1, W2, b2, W3, b3, W4, b4):
    batch = interaction_pairs.shape[0]
    d = table.shape[1]

    pair_table = _pack_pairs(table.T)

    flat = jnp.concatenate([interaction_pairs[:, 0], interaction_pairs[:, 1]])
    pair_idx = (flat // (2 * _B)) * _B + (flat % _B)
    halves = (interaction_pairs // _B) & 1

    gathered = _sc_gather_pairs(pair_table, pair_idx)

    blk = 2048
    nb = batch // blk
    grid = (nb,)
    full = lambda shape: pl.BlockSpec(shape, lambda i: (0, 0))
    out = pl.pallas_call(
        _mlp_body,
        grid=grid,
        in_specs=[
            pl.BlockSpec((blk, 2 * d), lambda i: (i, 0)),
            pl.BlockSpec((blk, 2 * d), lambda i, n=nb: (n + i, 0)),
            pl.BlockSpec((blk, 2), lambda i: (i, 0)),
            full(W1.shape),
            full((1, W1.shape[1])),
            full(W2.shape),
            full((1, W2.shape[1])),
            full(W3.shape),
            full((1, W3.shape[1])),
            full(W4.shape),
            full((1, 1)),
        ],
        out_specs=pl.BlockSpec((blk, 1), lambda i: (i, 0)),
        out_shape=jax.ShapeDtypeStruct((batch, 1), jnp.float32),
    )(gathered, gathered, halves, W1, b1.reshape(1, -1), W2,
      b2.reshape(1, -1), W3, b3.reshape(1, -1), W4, b4.reshape(1, 1))
    return out.reshape(batch)


# speedup vs baseline: 2.9797x; 1.0206x over previous
import functools

import jax
import jax.numpy as jnp
from jax import lax
from jax.experimental import pallas as pl
from jax.experimental.pallas import tpu as pltpu
from jax.experimental.pallas import tpu_sc as plsc

_NC = 2
_NS = 16
_NW = _NC * _NS
_B = 16384


def _pack_body(t_ref, o_ref):
    x = t_ref[...]
    b = x.shape[1] // 2
    y = jnp.concatenate([x[:, :b], x[:, b:]], axis=0)
    o_ref[...] = jnp.transpose(y)


def _pack_pairs(table_t):
    d, v = table_t.shape
    n_blocks = (v + 2 * _B - 1) // (2 * _B)
    n_rows = n_blocks * _B
    return pl.pallas_call(
        _pack_body,
        grid=(n_blocks,),
        in_specs=[pl.BlockSpec((d, 2 * _B), lambda k: (0, k))],
        out_specs=pl.BlockSpec((_B, 2 * d), lambda k: (k, 0)),
        out_shape=jax.ShapeDtypeStruct((n_rows, 2 * d), table_t.dtype),
    )(table_t)


def _sc_gather_pairs(pair_table, pair_idx):
    n_idx = pair_idx.shape[0]
    d = pair_table.shape[1]
    b_per_w = n_idx // _NW
    n_chunks = 4
    chunk = b_per_w // n_chunks
    mesh = plsc.VectorSubcoreMesh(core_axis_name="c", subcore_axis_name="s")

    @functools.partial(
        pl.kernel,
        mesh=mesh,
        out_type=jax.ShapeDtypeStruct((n_idx, d), pair_table.dtype),
        scratch_types=[
            pltpu.VMEM((b_per_w,), jnp.int32),
            pltpu.VMEM((chunk, d), pair_table.dtype),
            pltpu.VMEM((chunk, d), pair_table.dtype),
            pltpu.SemaphoreType.DMA,
            pltpu.SemaphoreType.DMA,
        ],
    )
    def gather_kernel(tab_hbm, idx_hbm, out_hbm, idx_v, rows_a, rows_b,
                      sem_a, sem_b):
        wid = lax.axis_index("s") * _NC + lax.axis_index("c")
        base = wid * b_per_w
        pltpu.sync_copy(idx_hbm.at[pl.ds(base, b_per_w)], idx_v)
        bufs = (rows_a, rows_b)
        sems = (sem_a, sem_b)
        cps = {}
        for i in range(2):
            cps[i] = pltpu.async_copy(
                tab_hbm.at[idx_v.at[pl.ds(i * chunk, chunk)]], bufs[i],
                sems[i])
        for i in range(n_chunks):
            cps[i].wait()
            pltpu.sync_copy(bufs[i % 2],
                            out_hbm.at[pl.ds(base + i * chunk, chunk)])
            if i + 2 < n_chunks:
                cps[i + 2] = pltpu.async_copy(
                    tab_hbm.at[idx_v.at[pl.ds((i + 2) * chunk, chunk)]],
                    bufs[i % 2], sems[i % 2])

    return gather_kernel(pair_table, pair_idx)


def _mlp_body(xu_ref, xv_ref, h_ref, w1_ref, b1_ref, w2_ref, b2_ref, w3_ref,
              b3_ref, w4_ref, b4_ref, o_ref):
    xu = xu_ref[...]
    xv = xv_ref[...]
    d = xu.shape[1] // 2
    hu = h_ref[:, 0:1]
    hv = h_ref[:, 1:2]
    u = jnp.where(hu == 0, xu[:, 0:d], xu[:, d:2 * d]).astype(jnp.float32)
    v = jnp.where(hv == 0, xv[:, 0:d], xv[:, d:2 * d]).astype(jnp.float32)
    mf = u * v
    mlp = jnp.concatenate([u, v], axis=1)
    h = jnp.maximum(
        jnp.dot(mlp, w1_ref[...], preferred_element_type=jnp.float32)
        + b1_ref[...], 0.0)
    h = jnp.maximum(
        jnp.dot(h, w2_ref[...], preferred_element_type=jnp.float32)
        + b2_ref[...], 0.0)
    h = jnp.maximum(
        jnp.dot(h, w3_ref[...], preferred_element_type=jnp.float32)
        + b3_ref[...], 0.0)
    nh = h.shape[1]
    out = (jnp.dot(h, w4_ref[:nh, :], preferred_element_type=jnp.float32)
           + jnp.dot(mf, w4_ref[nh:, :], preferred_element_type=jnp.float32)
           + b4_ref[...])
    o_ref[...] = out


def kernel(interaction_pairs, table, W1, b1, W2, b2, W3, b3, W4, b4):
    batch = interaction_pairs.shape[0]
    d = table.shape[1]

    pair_table = _pack_pairs(table.T)

    flat = jnp.concatenate([interaction_pairs[:, 0], interaction_pairs[:, 1]])
    pair_idx = (flat // (2 * _B)) * _B + (flat % _B)
    halves = (interaction_pairs // _B) & 1

    gathered = _sc_gather_pairs(pair_table, pair_idx)

    blk = 2048
    nb = batch // blk
    grid = (nb,)
    full = lambda shape: pl.BlockSpec(shape, lambda i: (0, 0))
    out = pl.pallas_call(
        _mlp_body,
        grid=grid,
        in_specs=[
            pl.BlockSpec((blk, 2 * d), lambda i: (i, 0)),
            pl.BlockSpec((blk, 2 * d), lambda i, n=nb: (n + i, 0)),
            pl.BlockSpec((blk, 2), lambda i: (i, 0)),
            full(W1.shape),
            full((1, W1.shape[1])),
            full(W2.shape),
            full((1, W2.shape[1])),
            full(W3.shape),
            full((1, W3.shape[1])),
            full(W4.shape),
            full((1, 1)),
        ],
        out_specs=pl.BlockSpec((blk, 1), lambda i: (i, 0)),
        out_shape=jax.ShapeDtypeStruct((batch, 1), jnp.float32),
    )(gathered, gathered, halves, W1, b1.reshape(1, -1), W2,
      b2.reshape(1, -1), W3, b3.reshape(1, -1), W4, b4.reshape(1, 1))
    return out.reshape(batch)


# speedup vs baseline: 2.9855x; 1.0019x over previous
import functools

import jax
import jax.numpy as jnp
from jax import lax
from jax.experimental import pallas as pl
from jax.experimental.pallas import tpu as pltpu
from jax.experimental.pallas import tpu_sc as plsc

_NC = 2
_NS = 16
_NW = _NC * _NS
_B = 16384


def _pack_body(t_ref, o_ref):
    x = t_ref[...]
    b = x.shape[1] // 2
    y = jnp.concatenate([x[:, :b], x[:, b:]], axis=0)
    o_ref[...] = jnp.transpose(y)


def _pack_pairs(table_t):
    d, v = table_t.shape
    n_blocks = (v + 2 * _B - 1) // (2 * _B)
    n_rows = n_blocks * _B
    return pl.pallas_call(
        _pack_body,
        grid=(n_blocks,),
        in_specs=[pl.BlockSpec((d, 2 * _B), lambda k: (0, k))],
        out_specs=pl.BlockSpec((_B, 2 * d), lambda k: (k, 0)),
        out_shape=jax.ShapeDtypeStruct((n_rows, 2 * d), table_t.dtype),
    )(table_t)


def _sc_gather_pairs(pair_table, pair_idx):
    n_idx = pair_idx.shape[0]
    d = pair_table.shape[1]
    b_per_w = n_idx // _NW
    n_chunks = 4
    chunk = b_per_w // n_chunks
    mesh = plsc.VectorSubcoreMesh(core_axis_name="c", subcore_axis_name="s")

    half_n = n_idx // 2

    @functools.partial(
        pl.kernel,
        mesh=mesh,
        out_type=(jax.ShapeDtypeStruct((half_n, d), pair_table.dtype),
                  jax.ShapeDtypeStruct((half_n, d), pair_table.dtype)),
        scratch_types=[
            pltpu.VMEM((b_per_w,), jnp.int32),
            pltpu.VMEM((chunk, d), pair_table.dtype),
            pltpu.VMEM((chunk, d), pair_table.dtype),
            pltpu.SemaphoreType.DMA,
            pltpu.SemaphoreType.DMA,
        ],
    )
    def gather_kernel(tab_hbm, idx_hbm, out_u, out_v, idx_v, rows_a, rows_b,
                      sem_a, sem_b):
        wid = lax.axis_index("s") * _NC + lax.axis_index("c")
        base = wid * b_per_w
        pltpu.sync_copy(idx_hbm.at[pl.ds(base, b_per_w)], idx_v)
        bufs = (rows_a, rows_b)
        sems = (sem_a, sem_b)
        cps = {}
        for i in range(2):
            cps[i] = pltpu.async_copy(
                tab_hbm.at[idx_v.at[pl.ds(i * chunk, chunk)]], bufs[i],
                sems[i])
        for i in range(n_chunks):
            cps[i].wait()

            @pl.when(base < half_n)
            def _(i=i):
                pltpu.sync_copy(
                    bufs[i % 2],
                    out_u.at[pl.ds(base + i * chunk, chunk)])

            @pl.when(base >= half_n)
            def _(i=i):
                pltpu.sync_copy(
                    bufs[i % 2],
                    out_v.at[pl.ds(base - half_n + i * chunk, chunk)])

            if i + 2 < n_chunks:
                cps[i + 2] = pltpu.async_copy(
                    tab_hbm.at[idx_v.at[pl.ds((i + 2) * chunk, chunk)]],
                    bufs[i % 2], sems[i % 2])

    return gather_kernel(pair_table, pair_idx)


def _mlp_body(xu_ref, xv_ref, h_ref, w1_ref, b1_ref, w2_ref, b2_ref, w3_ref,
              b3_ref, w4_ref, b4_ref, o_ref):
    xu = xu_ref[...]
    xv = xv_ref[...]
    d = xu.shape[1] // 2
    hu = h_ref[:, 0:1]
    hv = h_ref[:, 1:2]
    u = jnp.where(hu == 0, xu[:, 0:d], xu[:, d:2 * d]).astype(jnp.float32)
    v = jnp.where(hv == 0, xv[:, 0:d], xv[:, d:2 * d]).astype(jnp.float32)
    mf = u * v
    mlp = jnp.concatenate([u, v], axis=1)
    h = jnp.maximum(
        jnp.dot(mlp, w1_ref[...], preferred_element_type=jnp.float32)
        + b1_ref[...], 0.0)
    h = jnp.maximum(
        jnp.dot(h, w2_ref[...], preferred_element_type=jnp.float32)
        + b2_ref[...], 0.0)
    h = jnp.maximum(
        jnp.dot(h, w3_ref[...], preferred_element_type=jnp.float32)
        + b3_ref[...], 0.0)
    nh = h.shape[1]
    out = (jnp.dot(h, w4_ref[:nh, :], preferred_element_type=jnp.float32)
           + jnp.dot(mf, w4_ref[nh:, :], preferred_element_type=jnp.float32)
           + b4_ref[...])
    o_ref[...] = out.reshape(o_ref.shape)


def kernel(interaction_pairs, table, W1, b1, W2, b2, W3, b3, W4, b4):
    batch = interaction_pairs.shape[0]
    d = table.shape[1]

    pair_table = _pack_pairs(table.T)

    flat = jnp.concatenate([interaction_pairs[:, 0], interaction_pairs[:, 1]])
    pair_idx = (flat // (2 * _B)) * _B + (flat % _B)
    halves = (interaction_pairs // _B) & 1

    g_u, g_v = _sc_gather_pairs(pair_table, pair_idx)

    blk = 2048
    grid = (batch // blk,)
    full = lambda shape: pl.BlockSpec(shape, lambda i: (0, 0))
    out = pl.pallas_call(
        _mlp_body,
        grid=grid,
        in_specs=[
            pl.BlockSpec((blk, 2 * d), lambda i: (i, 0)),
            pl.BlockSpec((blk, 2 * d), lambda i: (i, 0)),
            pl.BlockSpec((blk, 2), lambda i: (i, 0)),
            full(W1.shape),
            full((1, W1.shape[1])),
            full(W2.shape),
            full((1, W2.shape[1])),
            full(W3.shape),
            full((1, W3.shape[1])),
            full(W4.shape),
            full((1, 1)),
        ],
        out_specs=pl.BlockSpec((blk,), lambda i: (i,)),
        out_shape=jax.ShapeDtypeStruct((batch,), jnp.float32),
    )(g_u, g_v, halves, W1, b1.reshape(1, -1), W2,
      b2.reshape(1, -1), W3, b3.reshape(1, -1), W4, b4.reshape(1, 1))
    return out


# speedup vs baseline: 2.9874x; 1.0006x over previous
import functools

import jax
import jax.numpy as jnp
from jax import lax
from jax.experimental import pallas as pl
from jax.experimental.pallas import tpu as pltpu
from jax.experimental.pallas import tpu_sc as plsc

_NC = 2
_NS = 16
_NW = _NC * _NS
_B = 16384


def _pack_body(t_ref, o_ref):
    x = t_ref[...]
    b = x.shape[1] // 2
    y = jnp.concatenate([x[:, :b], x[:, b:]], axis=0)
    o_ref[...] = jnp.transpose(y)


def _pack_pairs(table_t):
    d, v = table_t.shape
    n_blocks = (v + 2 * _B - 1) // (2 * _B)
    n_rows = n_blocks * _B
    return pl.pallas_call(
        _pack_body,
        grid=(n_blocks,),
        in_specs=[pl.BlockSpec((d, 2 * _B), lambda k: (0, k))],
        out_specs=pl.BlockSpec((_B, 2 * d), lambda k: (k, 0)),
        out_shape=jax.ShapeDtypeStruct((n_rows, 2 * d), table_t.dtype),
    )(table_t)


def _sc_gather_pairs(pair_table, pair_idx):
    n_idx = pair_idx.shape[0]
    d = pair_table.shape[1]
    b_per_w = n_idx // _NW
    n_chunks = 4
    chunk = b_per_w // n_chunks
    mesh = plsc.VectorSubcoreMesh(core_axis_name="c", subcore_axis_name="s")

    half_n = n_idx // 2

    @functools.partial(
        pl.kernel,
        mesh=mesh,
        out_type=(jax.ShapeDtypeStruct((half_n, d), pair_table.dtype),
                  jax.ShapeDtypeStruct((half_n, d), pair_table.dtype)),
        scratch_types=[
            pltpu.VMEM((b_per_w,), jnp.int32),
            pltpu.VMEM((chunk, d), pair_table.dtype),
            pltpu.VMEM((chunk, d), pair_table.dtype),
            pltpu.SemaphoreType.DMA,
            pltpu.SemaphoreType.DMA,
        ],
    )
    def gather_kernel(tab_hbm, idx_hbm, out_u, out_v, idx_v, rows_a, rows_b,
                      sem_a, sem_b):
        wid = lax.axis_index("s") * _NC + lax.axis_index("c")
        base = wid * b_per_w
        pltpu.sync_copy(idx_hbm.at[pl.ds(base, b_per_w)], idx_v)
        bufs = (rows_a, rows_b)
        sems = (sem_a, sem_b)
        cps = {}
        for i in range(2):
            cps[i] = pltpu.async_copy(
                tab_hbm.at[idx_v.at[pl.ds(i * chunk, chunk)]], bufs[i],
                sems[i])
        for i in range(n_chunks):
            cps[i].wait()

            @pl.when(base < half_n)
            def _(i=i):
                pltpu.sync_copy(
                    bufs[i % 2],
                    out_u.at[pl.ds(base + i * chunk, chunk)])

            @pl.when(base >= half_n)
            def _(i=i):
                pltpu.sync_copy(
                    bufs[i % 2],
                    out_v.at[pl.ds(base - half_n + i * chunk, chunk)])

            if i + 2 < n_chunks:
                cps[i + 2] = pltpu.async_copy(
                    tab_hbm.at[idx_v.at[pl.ds((i + 2) * chunk, chunk)]],
                    bufs[i % 2], sems[i % 2])

    return gather_kernel(pair_table, pair_idx)


def _mlp_body(xu_ref, xv_ref, h_ref, w1_ref, b1_ref, w2_ref, b2_ref, w3_ref,
              b3_ref, w4_ref, b4_ref, o_ref):
    xu = xu_ref[...]
    xv = xv_ref[...]
    d = xu.shape[1] // 2
    hu = h_ref[:, 0:1]
    hv = h_ref[:, 1:2]
    u = jnp.where(hu == 0, xu[:, 0:d], xu[:, d:2 * d])
    v = jnp.where(hv == 0, xv[:, 0:d], xv[:, d:2 * d])
    mf = u * v
    bf = jnp.bfloat16
    mlp = jnp.concatenate([u, v], axis=1).astype(bf)
    h = jnp.maximum(
        jnp.dot(mlp, w1_ref[...].astype(bf),
                preferred_element_type=jnp.float32) + b1_ref[...], 0.0)
    h = jnp.maximum(
        jnp.dot(h.astype(bf), w2_ref[...].astype(bf),
                preferred_element_type=jnp.float32) + b2_ref[...], 0.0)
    h = jnp.maximum(
        jnp.dot(h.astype(bf), w3_ref[...].astype(bf),
                preferred_element_type=jnp.float32) + b3_ref[...], 0.0)
    nh = h.shape[1]
    out = (jnp.dot(h.astype(bf), w4_ref[:nh, :].astype(bf),
                   preferred_element_type=jnp.float32)
           + jnp.dot(mf.astype(bf), w4_ref[nh:, :].astype(bf),
                     preferred_element_type=jnp.float32)
           + b4_ref[...])
    o_ref[...] = out.reshape(o_ref.shape)


def kernel(interaction_pairs, table, W1, b1, W2, b2, W3, b3, W4, b4):
    batch = interaction_pairs.shape[0]
    d = table.shape[1]

    pair_table = _pack_pairs(table.T)

    flat = jnp.concatenate([interaction_pairs[:, 0], interaction_pairs[:, 1]])
    pair_idx = (flat // (2 * _B)) * _B + (flat % _B)
    halves = (interaction_pairs // _B) & 1

    g_u, g_v = _sc_gather_pairs(pair_table, pair_idx)

    blk = 2048
    grid = (batch // blk,)
    full = lambda shape: pl.BlockSpec(shape, lambda i: (0, 0))
    out = pl.pallas_call(
        _mlp_body,
        grid=grid,
        in_specs=[
            pl.BlockSpec((blk, 2 * d), lambda i: (i, 0)),
            pl.BlockSpec((blk, 2 * d), lambda i: (i, 0)),
            pl.BlockSpec((blk, 2), lambda i: (i, 0)),
            full(W1.shape),
            full((1, W1.shape[1])),
            full(W2.shape),
            full((1, W2.shape[1])),
            full(W3.shape),
            full((1, W3.shape[1])),
            full(W4.shape),
            full((1, 1)),
        ],
        out_specs=pl.BlockSpec((blk,), lambda i: (i,)),
        out_shape=jax.ShapeDtypeStruct((batch,), jnp.float32),
    )(g_u, g_v, halves, W1, b1.reshape(1, -1), W2,
      b2.reshape(1, -1), W3, b3.reshape(1, -1), W4, b4.reshape(1, 1))
    return out
